# baseline (speedup 1.0000x reference)
import functools
import itertools

import numpy as np
import jax
import jax.numpy as jnp
from jax import lax
from jax.experimental import pallas as pl
from jax.experimental.pallas import tpu as pltpu
from jax.experimental.pallas import tpu_sc as plsc

HASH_SIZE = 2097152
NLEV = 8
GRIDS = np.round(np.geomspace(16, 2048, NLEV)).astype(np.int32)
N_PTS = 262144
NF = 2

NC = 2
NS = 16
L = 16
NW = NC * NS
PTS_PER_W = N_PTS // NW
C = 128
NCHUNK = PTS_PER_W // C
NG = C // L

_PI2 = np.int32(19349663)
_PI3 = np.int32(83492791)
_MASK = np.int32(HASH_SIZE - 1)


def _dim_weights(xs, level):
    gf = float(GRIDS[level])
    cw, fw = [], []
    for d in range(3):
        xn = (xs[d] + 2.0) * 0.25
        lo = xn * gf - 0.5
        iv = lo.astype(jnp.int32)
        fl = iv.astype(jnp.float32)
        cw.append(lo - fl)
        fw.append(1.0 - cw[-1])
    return cw, fw


def _corner_rows(xs, level):
    gf = float(GRIDS[level])
    ii = []
    for d in range(3):
        xn = (xs[d] + 2.0) * 0.25
        lo = xn * gf - 0.5
        ii.append(lo.astype(jnp.int32))
    b2 = ii[1] * _PI2
    b2c = b2 + _PI2
    b3 = ii[2] * _PI3
    b3c = b3 + _PI3
    x0 = ii[0]
    x0c = x0 + 1
    rows = []
    for cx, cy, cz in itertools.product((0, 1), repeat=2 + 1):
        h = (x0c if cx else x0) ^ (b2c if cy else b2) ^ (b3c if cz else b3)
        rows.append(h & _MASK)
    return rows


def _body(xt_hbm, *refs):
    tabs = refs[:NLEV * NF]
    out_hbm = refs[NLEV * NF]
    xch, idxb, gbuf, och, sem = refs[NLEV * NF + 1:]
    wid = lax.axis_index("s") * NC + lax.axis_index("c")
    base0 = wid * PTS_PER_W

    def chunk_body(ci, carry):
        base = base0 + ci * C
        pltpu.sync_copy(xt_hbm.at[:, pl.ds(base, C)], xch)

        def group_a(g, c2):
            p = g * L
            xs = [xch[d, pl.ds(p, L)] for d in range(3)]
            for lev in range(NLEV):
                rows = _corner_rows(xs, lev)
                for cn in range(8):
                    lc = lev * 8 + cn
                    idxb[pl.ds(lc * C + p, L)] = rows[cn]
            return c2

        lax.fori_loop(0, NG, group_a, 0)

        copies = []
        for lev in range(NLEV):
            isl = idxb.at[pl.ds(lev * 8 * C, 8 * C)]
            for f in range(NF):
                copies.append(
                    pltpu.async_copy(
                        tabs[lev * NF + f].at[isl],
                        gbuf.at[pl.ds((lev * NF + f) * 8 * C, 8 * C)], sem))
        for cp in copies:
            cp.wait()

        def group_b(g, c2):
            p = g * L
            xs = [xch[d, pl.ds(p, L)] for d in range(3)]
            for lev in range(NLEV):
                cw, fw = _dim_weights(xs, lev)
                wyz = {}
                for cy in range(2):
                    for cz in range(2):
                        wyz[(cy, cz)] = ((fw[1] if cy == 0 else cw[1])
                                         * (fw[2] if cz == 0 else cw[2]))
                acc0 = None
                acc1 = None
                for ci2, (cx, cy, cz) in enumerate(
                        itertools.product((0, 1), repeat=3)):
                    w = (fw[0] if cx == 0 else cw[0]) * wyz[(cy, cz)]
                    g0 = gbuf[pl.ds((lev * NF * 8 + ci2) * C + p, L)]
                    g1 = gbuf[pl.ds(((lev * NF + 1) * 8 + ci2) * C + p, L)]
                    acc0 = w * g0 if acc0 is None else acc0 + w * g0
                    acc1 = w * g1 if acc1 is None else acc1 + w * g1
                och[2 * lev, pl.ds(p, L)] = acc0 * 10.0
                och[2 * lev + 1, pl.ds(p, L)] = acc1 * 10.0
            return c2

        lax.fori_loop(0, NG, group_b, 0)

        pltpu.sync_copy(och, out_hbm.at[:, pl.ds(base, C)])
        return carry

    lax.fori_loop(0, NCHUNK, chunk_body, 0)


def _run(xt, *tabs):
    mesh = plsc.VectorSubcoreMesh(core_axis_name="c", subcore_axis_name="s")
    f = pl.kernel(
        _body,
        out_type=jax.ShapeDtypeStruct((2 * NLEV, N_PTS), jnp.float32),
        mesh=mesh,
        scratch_types=[
            pltpu.VMEM((3, C), jnp.float32),
            pltpu.VMEM((NLEV * 8 * C,), jnp.int32),
            pltpu.VMEM((NLEV * NF * 8 * C,), jnp.float32),
            pltpu.VMEM((2 * NLEV, C), jnp.float32),
            pltpu.SemaphoreType.DMA,
        ],
    )
    return f(xt, *tabs)


@jax.jit
def _encode(x, hash_tables):
    xt = x.T
    tabs = [hash_tables[i, :, f] for i in range(NLEV) for f in range(NF)]
    out = _run(xt, *tabs)
    return out.T


def kernel(x, hash_tables):
    return _encode(x, hash_tables)


# speedup vs baseline: 1.0021x; 1.0021x over previous
import functools
import itertools

import numpy as np
import jax
import jax.numpy as jnp
from jax import lax
from jax.experimental import pallas as pl
from jax.experimental.pallas import tpu as pltpu
from jax.experimental.pallas import tpu_sc as plsc

HASH_SIZE = 2097152
NLEV = 8
GRIDS = np.round(np.geomspace(16, 2048, NLEV)).astype(np.int32)
N_PTS = 262144
NF = 2

NC = 2
NS = 16
L = 16
NW = NC * NS
PTS_PER_W = N_PTS // NW
C = 128
NCHUNK = PTS_PER_W // C
NG = C // L

_PI2 = np.int32(19349663)
_PI3 = np.int32(83492791)
_MASK = np.int32(HASH_SIZE - 1)


def _level_coords(xs, level):
    gf = float(GRIDS[level])
    ii = []
    for d in range(3):
        xn = (xs[d] + 2.0) * 0.25
        lo = xn * gf - 0.5
        ii.append(lo.astype(jnp.int32))
    return ii


def _dim_weights(xs, level):
    gf = float(GRIDS[level])
    cw, fw = [], []
    for d in range(3):
        xn = (xs[d] + 2.0) * 0.25
        lo = xn * gf - 0.5
        iv = lo.astype(jnp.int32)
        fl = iv.astype(jnp.float32)
        cw.append(lo - fl)
        fw.append(1.0 - cw[-1])
    return cw, fw


def _body(xt_hbm, *refs):
    tabs = refs[:NLEV * NF]
    out_hbm = refs[NLEV * NF]
    (xch0, xch1, idx0, idx1, gb0, gb1, och, sem0, sem1) = refs[NLEV * NF + 1:]

    wid = lax.axis_index("s") * NC + lax.axis_index("c")
    base0 = wid * PTS_PER_W

    def pass_a(base, xch, idxb):
        pltpu.sync_copy(xt_hbm.at[:, pl.ds(base, C)], xch)

        def group_a(g, c2):
            p = g * L
            xs = [xch[d, pl.ds(p, L)] for d in range(3)]
            for lev in range(NLEV):
                ii = _level_coords(xs, lev)
                b2 = ii[1] * _PI2
                b2c = b2 + _PI2
                b3 = ii[2] * _PI3
                b3c = b3 + _PI3
                x0 = ii[0]
                x0c = x0 + 1
                for cn, (qx, qy, qz) in enumerate(itertools.product((0, 1), repeat=3)):
                    h = ((x0c if qx else x0) ^ (b2c if qy else b2)
                         ^ (b3c if qz else b3)) & _MASK
                    idxb[pl.ds((lev * 8 + cn) * C + p, L)] = h
            return c2

        lax.fori_loop(0, NG, group_a, 0)

    def fire(idxb, gbuf, sem):
        copies = []
        for lev in range(NLEV):
            isl = idxb.at[pl.ds(lev * 8 * C, 8 * C)]
            for f in range(NF):
                copies.append(
                    pltpu.async_copy(
                        tabs[lev * NF + f].at[isl],
                        gbuf.at[pl.ds((lev * NF + f) * 8 * C, 8 * C)], sem))
        return copies

    def pass_b(base, xch, gbuf):
        def group_b(g, c2):
            p = g * L
            xs = [xch[d, pl.ds(p, L)] for d in range(3)]
            for lev in range(NLEV):
                cw, fw = _dim_weights(xs, lev)
                wyz = {}
                for qy in range(2):
                    for qz in range(2):
                        wyz[(qy, qz)] = ((fw[1] if qy == 0 else cw[1])
                                         * (fw[2] if qz == 0 else cw[2]))
                acc0 = None
                acc1 = None
                for ci2, (qx, qy, qz) in enumerate(
                        itertools.product((0, 1), repeat=3)):
                    w = (fw[0] if qx == 0 else cw[0]) * wyz[(qy, qz)]
                    g0 = gbuf[pl.ds((lev * NF * 8 + ci2) * C + p, L)]
                    g1 = gbuf[pl.ds(((lev * NF + 1) * 8 + ci2) * C + p, L)]
                    acc0 = w * g0 if acc0 is None else acc0 + w * g0
                    acc1 = w * g1 if acc1 is None else acc1 + w * g1
                och[2 * lev, pl.ds(p, L)] = acc0 * 10.0
                och[2 * lev + 1, pl.ds(p, L)] = acc1 * 10.0
            return c2

        lax.fori_loop(0, NG, group_b, 0)
        pltpu.sync_copy(och, out_hbm.at[:, pl.ds(base, C)])

    def pair_body(cp, carry):
        b0 = base0 + (2 * cp) * C
        b1 = b0 + C
        pass_a(b0, xch0, idx0)
        cps0 = fire(idx0, gb0, sem0)
        pass_a(b1, xch1, idx1)
        cps1 = fire(idx1, gb1, sem1)
        for c in cps0:
            c.wait()
        pass_b(b0, xch0, gb0)
        for c in cps1:
            c.wait()
        pass_b(b1, xch1, gb1)
        return carry

    lax.fori_loop(0, NCHUNK // 2, pair_body, 0)


def _run(xt, *tabs):
    mesh = plsc.VectorSubcoreMesh(core_axis_name="c", subcore_axis_name="s")
    f = pl.kernel(
        _body,
        out_type=jax.ShapeDtypeStruct((2 * NLEV, N_PTS), jnp.float32),
        mesh=mesh,
        scratch_types=[
            pltpu.VMEM((3, C), jnp.float32),
            pltpu.VMEM((3, C), jnp.float32),
            pltpu.VMEM((NLEV * 8 * C,), jnp.int32),
            pltpu.VMEM((NLEV * 8 * C,), jnp.int32),
            pltpu.VMEM((NLEV * NF * 8 * C,), jnp.float32),
            pltpu.VMEM((NLEV * NF * 8 * C,), jnp.float32),
            pltpu.VMEM((2 * NLEV, C), jnp.float32),
            pltpu.SemaphoreType.DMA,
            pltpu.SemaphoreType.DMA,
        ],
    )
    return f(xt, *tabs)


@jax.jit
def _encode(x, hash_tables):
    xt = x.T
    tabs = [hash_tables[i, :, f] for i in range(NLEV) for f in range(NF)]
    out = _run(xt, *tabs)
    return out.T


def kernel(x, hash_tables):
    return _encode(x, hash_tables)
